# baseline (speedup 1.0000x reference)
import jax
import jax.numpy as jnp
from jax import lax
from jax.experimental import pallas as pl
from jax.experimental.pallas import tpu as pltpu
from jax.experimental.pallas import tpu_sc as plsc

D = 64
B = 16384
NC = 2
NS = 16
NW = NC * NS
B_PER_W = B // NW
CHUNK = 128
NCHUNK = B_PER_W // CHUNK
L = 16
NG = B_PER_W // L


def _embed_body(table_hbm, tokens_hbm, out_hbm, tok_v, i0, i1, i2, i3,
                rows_v, out_v, sem):
    idx_refs = [i0, i1, i2, i3]
    wid = lax.axis_index("s") * NC + lax.axis_index("c")
    base = wid * B_PER_W
    pltpu.sync_copy(tokens_hbm.at[wid], tok_v)
    for j in range(NCHUNK):
        for i in range(CHUNK // L):
            s = pl.ds(i * L, L)
            idx_refs[j][s] = jax.lax.shift_right_logical(
                tok_v[pl.ds(j * CHUNK + i * L, L)], 1)
    lanes = jax.lax.broadcasted_iota(jnp.int32, (L,), 0)
    for j in range(NCHUNK):
        pltpu.async_copy(table_hbm.at[idx_refs[j]], rows_v, sem).wait()
        for i in range(CHUNK // L):
            tok = tok_v[pl.ds(j * CHUNK + i * L, L)]
            half = (tok & 1) * D
            rowloc = lanes + i * L
            outrow = rowloc + j * CHUNK

            def extract(c, _):
                vals = plsc.load_gather(rows_v, [rowloc, half + c])
                plsc.store_scatter(
                    out_v, [outrow, jnp.full((L,), c, jnp.int32)], vals)
                return 0

            lax.fori_loop(0, D, extract, 0)
    pltpu.sync_copy(out_v, out_hbm.at[pl.ds(base, B_PER_W)])


@jax.jit
def _embed(table2, tokens2):
    call = pl.kernel(
        _embed_body,
        out_type=jax.ShapeDtypeStruct((B, D), jnp.float32),
        mesh=plsc.VectorSubcoreMesh(core_axis_name="c", subcore_axis_name="s"),
        scratch_types=[
            pltpu.VMEM((B_PER_W,), jnp.int32),
            pltpu.VMEM((CHUNK,), jnp.int32),
            pltpu.VMEM((CHUNK,), jnp.int32),
            pltpu.VMEM((CHUNK,), jnp.int32),
            pltpu.VMEM((CHUNK,), jnp.int32),
            pltpu.VMEM((CHUNK, 2 * D), jnp.float32),
            pltpu.VMEM((B_PER_W, D), jnp.float32),
            pltpu.SemaphoreType.DMA,
        ],
        compiler_params=pltpu.CompilerParams(needs_layout_passes=False),
    )
    return call(table2, tokens2)


def kernel(tokens, embedding_weight):
    table2 = embedding_weight.reshape(500000, 2 * D)
    tokens2 = tokens.astype(jnp.int32).reshape(NW, B_PER_W)
    return _embed(table2, tokens2)


# speedup vs baseline: 2.7112x; 2.7112x over previous
import jax
import jax.numpy as jnp
from jax import lax
from jax.experimental import pallas as pl
from jax.experimental.pallas import tpu as pltpu
from jax.experimental.pallas import tpu_sc as plsc

D = 64
B = 16384
NC = 2
NS = 16
NW = NC * NS
B_PER_W = B // NW
L = 16
NG = B_PER_W // L
G = B // 8


def _embed_body(table_hbm, tokens_hbm, out_hbm, tok_v, out_v, sem):
    wid = lax.axis_index("s") * NC + lax.axis_index("c")
    pltpu.sync_copy(tokens_hbm.at[wid], tok_v)

    def issue_group(g, carry):
        tok = tok_v[pl.ds(g * L, L)]
        grp = jax.lax.shift_right_logical(tok, 3)
        sub = tok & 7
        for lane in range(L):
            pltpu.async_copy(
                table_hbm.at[grp[lane], sub[lane]],
                out_v.at[2 * g + (lane >> 3), lane & 7],
                sem,
            )
        return carry

    lax.fori_loop(0, NG, issue_group, 0)
    pltpu.make_async_copy(table_hbm.at[pl.ds(0, 2 * NG)], out_v, sem).wait()
    pltpu.sync_copy(out_v, out_hbm.at[pl.ds(wid * 2 * NG, 2 * NG)])


@jax.jit
def _embed(table3, tokens2):
    call = pl.kernel(
        _embed_body,
        out_type=jax.ShapeDtypeStruct((G, 8, D), jnp.float32),
        mesh=plsc.VectorSubcoreMesh(core_axis_name="c", subcore_axis_name="s"),
        scratch_types=[
            pltpu.VMEM((B_PER_W,), jnp.int32),
            pltpu.VMEM((2 * NG, 8, D), jnp.float32),
            pltpu.SemaphoreType.DMA,
        ],
        compiler_params=pltpu.CompilerParams(needs_layout_passes=False),
    )
    return call(table3, tokens2)


def kernel(tokens, embedding_weight):
    table3 = embedding_weight.reshape(125000, 8, D)
    tokens2 = tokens.astype(jnp.int32).reshape(NW, B_PER_W)
    out3 = _embed(table3, tokens2)
    return out3.reshape(B, D)
